# baseline (speedup 1.0000x reference)
import functools

import jax
import jax.numpy as jnp
from jax import lax
from jax.experimental import pallas as pl
from jax.experimental.pallas import tpu as pltpu
from jax.experimental.pallas import tpu_sc as plsc

B = 4096
D = 1024
K = 8192
S = 16

KT = 512
NK = K // KT

NW = 32
CH = 32

_MESH = dict(core_axis_name="c", subcore_axis_name="s",
             num_cores=2, num_subcores=16)


def _prep_body(w_ref, wb_ref, wt_ref):
    w = w_ref[...]
    wb_ref[...] = w.astype(jnp.bfloat16)
    wt_ref[...] = w.T


def _prep(W):
    return pl.pallas_call(
        _prep_body,
        grid=(NK,),
        in_specs=[pl.BlockSpec((D, KT), lambda k: (0, k))],
        out_specs=[
            pl.BlockSpec((D, KT), lambda k: (0, k)),
            pl.BlockSpec((KT, D), lambda k: (k, 0)),
        ],
        out_shape=[
            jax.ShapeDtypeStruct((D, K), jnp.bfloat16),
            jax.ShapeDtypeStruct((K, D), jnp.float32),
        ],
    )(W)


def _tc_step(r, Wb):
    nb = r.shape[0]

    def body(r_ref, wb_ref, j_ref, z_ref, rb_scr, c_scr, smax_scr,
             sidx_scr, sval_scr):
        k = pl.program_id(0)

        @pl.when(k == 0)
        def _():
            rb_scr[...] = r_ref[...].astype(jnp.bfloat16)
            smax_scr[...] = jnp.full((nb, 1), -1.0, jnp.float32)
            sidx_scr[...] = jnp.zeros((nb, 1), jnp.int32)
            sval_scr[...] = jnp.zeros((nb, 1), jnp.float32)

        c_scr[k % 2] = jnp.dot(rb_scr[...], wb_ref[...],
                               preferred_element_type=jnp.float32)

        c = c_scr[(k + 1) % 2]
        a = jnp.abs(c)
        lmax = jnp.max(a, axis=1, keepdims=True)
        ii = lax.broadcasted_iota(jnp.int32, (nb, KT), 1)
        lidx = jnp.min(jnp.where(a == lmax, ii, KT), axis=1, keepdims=True)
        lval = jnp.sum(jnp.where(ii == lidx, c, 0.0), axis=1, keepdims=True)

        upd = jnp.logical_and(lmax > smax_scr[...], k > 0)
        smax_scr[...] = jnp.where(upd, lmax, smax_scr[...])
        sidx_scr[...] = jnp.where(upd, lidx + (k - 1) * KT, sidx_scr[...])
        sval_scr[...] = jnp.where(upd, lval, sval_scr[...])

        @pl.when(k == NK)
        def _():
            j_ref[...] = sidx_scr[...]
            z_ref[...] = sval_scr[...]

    return pl.pallas_call(
        body,
        grid=(NK + 1,),
        in_specs=[
            pl.BlockSpec((nb, D), lambda k: (0, 0)),
            pl.BlockSpec((D, KT), lambda k: (0, jnp.minimum(k, NK - 1))),
        ],
        out_specs=[
            pl.BlockSpec((nb, 1), lambda k: (0, 0)),
            pl.BlockSpec((nb, 1), lambda k: (0, 0)),
        ],
        out_shape=[
            jax.ShapeDtypeStruct((nb, 1), jnp.int32),
            jax.ShapeDtypeStruct((nb, 1), jnp.float32),
        ],
        scratch_shapes=[
            pltpu.VMEM((nb, D), jnp.bfloat16),
            pltpu.VMEM((2, nb, KT), jnp.float32),
            pltpu.VMEM((nb, 1), jnp.float32),
            pltpu.VMEM((nb, 1), jnp.int32),
            pltpu.VMEM((nb, 1), jnp.float32),
        ],
    )(r, Wb)


def _bf16r(v):
    u = lax.bitcast_convert_type(v, jnp.uint32)
    u = (u + jnp.uint32(0x7FFF) + ((u >> 16) & jnp.uint32(1))) \
        & jnp.uint32(0xFFFF0000)
    return lax.bitcast_convert_type(u, jnp.float32)


def _chunk_update(z_c, sel_v, r_v, xh_v):
    for g in range(CH // 16):
        zvec = z_c[pl.ds(g * 16, 16)]
        zvb = _bf16r(zvec)
        for rr in range(16):
            i = g * 16 + rr
            zz = zvec[rr]
            zb = zvb[rr]

            def col_body(t, _, i=i, zz=zz, zb=zb):
                off = t * 64
                for u in range(4):
                    sl = pl.ds(off + u * 16, 16)
                    s = sel_v[i, sl]
                    r_v[i, sl] = r_v[i, sl] - zz * s
                    xh_v[i, sl] = xh_v[i, sl] + zb * _bf16r(s)
                return 0

            lax.fori_loop(0, D // 64, col_body, 0)


def _sc_update(j, z, r, xh, WT):
    nb = r.shape[0]
    rpw = nb // NW
    nch = rpw // CH

    def body(j_hbm, z_hbm, r_hbm, xh_hbm, wt_hbm, out_hbm, xout_hbm,
             idx_c, z_c, sel_v, r_v, xh_v, sem):
        wid = lax.axis_index("s") * 2 + lax.axis_index("c")
        base = wid * rpw

        def chunk_body(ci, _):
            rows0 = base + ci * CH
            pltpu.sync_copy(j_hbm.at[pl.ds(rows0, CH)], idx_c)
            pltpu.async_copy(wt_hbm.at[idx_c], sel_v, sem).wait()
            pltpu.sync_copy(z_hbm.at[pl.ds(rows0, CH)], z_c)
            pltpu.sync_copy(r_hbm.at[pl.ds(rows0, CH)], r_v)
            pltpu.sync_copy(xh_hbm.at[pl.ds(rows0, CH)], xh_v)
            _chunk_update(z_c, sel_v, r_v, xh_v)
            pltpu.sync_copy(r_v, out_hbm.at[pl.ds(rows0, CH)])
            pltpu.sync_copy(xh_v, xout_hbm.at[pl.ds(rows0, CH)])
            return 0

        lax.fori_loop(0, nch, chunk_body, 0)

    fn = pl.kernel(
        body,
        out_type=[jax.ShapeDtypeStruct((nb, D), jnp.float32),
                  jax.ShapeDtypeStruct((nb, D), jnp.float32)],
        mesh=plsc.VectorSubcoreMesh(**_MESH),
        scratch_types=[
            pltpu.VMEM((CH,), jnp.int32),
            pltpu.VMEM((CH,), jnp.float32),
            pltpu.VMEM((CH, D), jnp.float32),
            pltpu.VMEM((CH, D), jnp.float32),
            pltpu.VMEM((CH, D), jnp.float32),
            pltpu.SemaphoreType.DMA,
        ],
    )
    return fn(j, z, r, xh, WT)


def _sc_final(j, z, xh, b_dec, WT):
    nb = xh.shape[0]
    rpw = nb // NW
    nch = rpw // CH

    def body(j_hbm, z_hbm, xh_hbm, bd_hbm, wt_hbm, out_hbm,
             idx_c, z_c, sel_v, xh_v, bd_v, sem):
        wid = lax.axis_index("s") * 2 + lax.axis_index("c")
        base = wid * rpw
        pltpu.sync_copy(bd_hbm, bd_v)

        def chunk_body(ci, _):
            rows0 = base + ci * CH
            pltpu.sync_copy(j_hbm.at[pl.ds(rows0, CH)], idx_c)
            pltpu.async_copy(wt_hbm.at[idx_c], sel_v, sem).wait()
            pltpu.sync_copy(z_hbm.at[pl.ds(rows0, CH)], z_c)
            pltpu.sync_copy(xh_hbm.at[pl.ds(rows0, CH)], xh_v)

            for g in range(CH // 16):
                zvec = z_c[pl.ds(g * 16, 16)]
                zvb = _bf16r(zvec)
                for rr in range(16):
                    i = g * 16 + rr
                    zb = zvb[rr]

                    def col_body(t, _, i=i, zb=zb):
                        off = t * 64
                        for u in range(4):
                            sl = pl.ds(off + u * 16, 16)
                            xh_v[i, sl] = (xh_v[i, sl]
                                           + zb * _bf16r(sel_v[i, sl])
                                           + bd_v[sl])
                        return 0

                    lax.fori_loop(0, D // 64, col_body, 0)

            pltpu.sync_copy(xh_v, out_hbm.at[pl.ds(rows0, CH)])
            return 0

        lax.fori_loop(0, nch, chunk_body, 0)

    fn = pl.kernel(
        body,
        out_type=jax.ShapeDtypeStruct((nb, D), jnp.float32),
        mesh=plsc.VectorSubcoreMesh(**_MESH),
        scratch_types=[
            pltpu.VMEM((CH,), jnp.int32),
            pltpu.VMEM((CH,), jnp.float32),
            pltpu.VMEM((CH, D), jnp.float32),
            pltpu.VMEM((CH, D), jnp.float32),
            pltpu.VMEM((D,), jnp.float32),
            pltpu.SemaphoreType.DMA,
        ],
    )
    return fn(j, z, xh, b_dec, WT)


def kernel(x, W, b_dec):
    Wb, WT = _prep(W)
    r = x - b_dec
    zero = jnp.zeros_like(x)
    rs = [r[:B // 2], r[B // 2:]]
    acc = [zero[:B // 2], zero[B // 2:]]
    xh = [None, None]
    for s in range(S):
        picks = []
        for h in range(2):
            jh, zh = _tc_step(rs[h], Wb)
            picks.append((jh.reshape(-1), zh.reshape(-1)))
        for h in range(2):
            jf, zf = picks[h]
            if s < S - 1:
                rs[h], acc[h] = _sc_update(jf, zf, rs[h], acc[h], WT)
            else:
                xh[h] = _sc_final(jf, zf, acc[h], b_dec, WT)
    return jnp.concatenate(xh, axis=0)
